# baseline (speedup 1.0000x reference)
import functools

import jax
import jax.numpy as jnp
from jax import lax
from jax.experimental import pallas as pl
from jax.experimental.pallas import tpu as pltpu
from jax.experimental.pallas import tpu_sc as plsc


def _make_broadcast_kernel(batch, seq_len, d_model, dtype):
    info = plsc.get_sparse_core_info()
    nw = info.num_cores * info.num_subcores
    assert seq_len % nw == 0
    rows_per_w = seq_len // nw

    mesh = plsc.VectorSubcoreMesh(core_axis_name="c", subcore_axis_name="s")

    rows_per_sc = seq_len // info.num_cores

    @functools.partial(
        pl.kernel,
        mesh=mesh,
        out_type=jax.ShapeDtypeStruct((batch, seq_len, d_model), dtype),
        scratch_types=[
            pltpu.VMEM_SHARED((rows_per_sc, d_model), dtype),
        ]
        + [pltpu.SemaphoreType.DMA] * batch,
    )
    def broadcast_kernel(pos_hbm, out_hbm, shared, *sems):
        c = lax.axis_index("c")
        s = lax.axis_index("s")
        wid = s * info.num_cores + c
        base = wid * rows_per_w
        slot = s * rows_per_w
        pltpu.sync_copy(pos_hbm.at[pl.ds(base, rows_per_w)], shared.at[pl.ds(slot, rows_per_w)])
        copies = [
            pltpu.async_copy(
                shared.at[pl.ds(slot, rows_per_w)],
                out_hbm.at[b, pl.ds(base, rows_per_w)],
                sems[b],
            )
            for b in range(batch)
        ]
        for cp in copies:
            cp.wait()

    return broadcast_kernel


def kernel(x, pos_table):
    batch, seq_len = x.shape[0], x.shape[1]
    d_model = pos_table.shape[1]
    fn = _make_broadcast_kernel(batch, seq_len, d_model, pos_table.dtype)
    return fn(pos_table[:seq_len])


# speedup vs baseline: 1.0969x; 1.0969x over previous
import functools

import jax
import jax.numpy as jnp
from jax import lax
from jax.experimental import pallas as pl
from jax.experimental.pallas import tpu as pltpu
from jax.experimental.pallas import tpu_sc as plsc


def _make_broadcast_kernel(batch, seq_len, d_model, dtype):
    info = plsc.get_sparse_core_info()
    nw = info.num_cores * info.num_subcores
    assert seq_len % nw == 0
    rows_per_w = seq_len // nw

    mesh = plsc.VectorSubcoreMesh(core_axis_name="c", subcore_axis_name="s")

    rows_per_sc = seq_len // info.num_cores

    @functools.partial(
        pl.kernel,
        mesh=mesh,
        out_type=jax.ShapeDtypeStruct((batch, seq_len, d_model), dtype),
        scratch_types=[
            pltpu.VMEM((rows_per_w, d_model), dtype),
            pltpu.VMEM_SHARED((rows_per_sc, d_model), dtype),
            pltpu.SemaphoreType.DMA,
            pltpu.SemaphoreType.DMA,
            pltpu.SemaphoreType.DMA,
            pltpu.SemaphoreType.DMA,
        ],
    )
    def broadcast_kernel(pos_hbm, out_hbm, buf_v, shared, lsem, ssem, wsem, wsem2):
        c = lax.axis_index("c")
        s = lax.axis_index("s")
        wid = s * info.num_cores + c
        base = wid * rows_per_w
        slot = s * rows_per_w
        rows = pl.ds(base, rows_per_w)
        l1 = pltpu.async_copy(pos_hbm.at[rows], buf_v, lsem)
        l2 = pltpu.async_copy(pos_hbm.at[rows], shared.at[pl.ds(slot, rows_per_w)], ssem)
        l1.wait()
        writes = [
            pltpu.async_copy(buf_v, out_hbm.at[b, rows], wsem)
            for b in range(batch - 1)
        ]
        l2.wait()
        w_last = pltpu.async_copy(
            shared.at[pl.ds(slot, rows_per_w)], out_hbm.at[batch - 1, rows], wsem2
        )
        for w in writes:
            w.wait()
        w_last.wait()

    return broadcast_kernel


def kernel(x, pos_table):
    batch, seq_len = x.shape[0], x.shape[1]
    d_model = pos_table.shape[1]
    fn = _make_broadcast_kernel(batch, seq_len, d_model, pos_table.dtype)
    return fn(pos_table[:seq_len])


# speedup vs baseline: 1.1849x; 1.0802x over previous
import functools

import jax
import jax.numpy as jnp
from jax import lax
from jax.experimental import pallas as pl
from jax.experimental.pallas import tpu as pltpu
from jax.experimental.pallas import tpu_sc as plsc


def _make_broadcast_kernel(batch, seq_len, d_model, dtype):
    info = plsc.get_sparse_core_info()
    nw = info.num_cores * info.num_subcores
    assert seq_len % nw == 0
    rows_per_w = seq_len // nw

    mesh = plsc.VectorSubcoreMesh(core_axis_name="c", subcore_axis_name="s")

    @functools.partial(
        pl.kernel,
        mesh=mesh,
        out_type=jax.ShapeDtypeStruct((batch, seq_len, d_model), dtype),
        scratch_types=[
            pltpu.VMEM((rows_per_w, d_model), dtype),
            pltpu.SemaphoreType.DMA,
        ],
    )
    def broadcast_kernel(pos_hbm, out_hbm, buf_v, wsem):
        wid = lax.axis_index("s") * info.num_cores + lax.axis_index("c")
        base = wid * rows_per_w
        rows = pl.ds(base, rows_per_w)
        pltpu.sync_copy(pos_hbm.at[rows], buf_v)
        writes = [
            pltpu.async_copy(buf_v, out_hbm.at[b, rows], wsem) for b in range(batch)
        ]
        for w in writes:
            w.wait()

    return broadcast_kernel


def kernel(x, pos_table):
    batch, seq_len = x.shape[0], x.shape[1]
    d_model = pos_table.shape[1]
    fn = _make_broadcast_kernel(batch, seq_len, d_model, pos_table.dtype)
    return fn(pos_table[:seq_len])
